# baseline (speedup 1.0000x reference)
import functools
import jax
import jax.numpy as jnp
from jax import lax
from jax.experimental import pallas as pl
from jax.experimental.pallas import tpu as pltpu
from jax.experimental.pallas import tpu_sc as plsc

_B = 4096
_F = 26
_CARD = 1000
_W = _F * _CARD
_NC = 2
_NS = 16
_NW = _NC * _NS
_BPW = _B // _NW
_RPW = _BPW * _F
_NB = 4
_NCHUNK = _BPW // _NB
_ROWS_PER_CHUNK = _NB * _F
_GROUP_STARTS = (0, 16, 32, 48, 64, 80, 88)


@functools.partial(
    pl.kernel,
    mesh=plsc.VectorSubcoreMesh(core_axis_name="c", subcore_axis_name="s"),
    out_type=jax.ShapeDtypeStruct((_B, _W), jnp.float32),
    scratch_types=[
        pltpu.VMEM((_RPW,), jnp.int32),
        pltpu.VMEM((_NB, _W), jnp.float32),
    ],
    compiler_params=pltpu.CompilerParams(
        use_tc_tiling_on_sc=False, needs_layout_passes=False
    ),
)
def _sc_onehot(xe_hbm, out_hbm, idx_v, buf):
    wid = lax.axis_index("s") * _NC + lax.axis_index("c")
    base_row = wid * _RPW
    pltpu.sync_copy(xe_hbm.at[pl.ds(base_row, _RPW)], idx_v)

    zeros16 = jnp.zeros((16,), jnp.float32)
    ones16 = jnp.ones((16,), jnp.float32)
    riota = lax.iota(jnp.int32, 16)

    def zbody(i, carry):
        r = i // (_W // 16)
        k = i % (_W // 16)
        buf[r, pl.ds(pl.multiple_of(k * 16, 16), 16)] = zeros16
        return carry

    lax.fori_loop(0, _NB * (_W // 16), zbody, 0)

    def chunk_body(c, carry):
        for g in _GROUP_STARTS:
            vals = idx_v[pl.ds(c * _ROWS_PER_CHUNK + g, 16)]
            rloc = g + riota
            rows = rloc // _F
            cols = (rloc % _F) * _CARD + vals
            plsc.store_scatter(buf, [rows, cols], ones16)
        pltpu.sync_copy(
            buf, out_hbm.at[pl.ds(wid * _BPW + c * _NB, _NB), :]
        )
        for g in _GROUP_STARTS:
            vals = idx_v[pl.ds(c * _ROWS_PER_CHUNK + g, 16)]
            rloc = g + riota
            rows = rloc // _F
            cols = (rloc % _F) * _CARD + vals
            plsc.store_scatter(buf, [rows, cols], zeros16)
        return carry

    lax.fori_loop(0, _NCHUNK, chunk_body, 0)


def kernel(xe):
    return _sc_onehot(xe.reshape(-1))
